# baseline (speedup 1.0000x reference)
import functools

import jax
import jax.numpy as jnp
from jax import lax
from jax.experimental import pallas as pl
from jax.experimental.pallas import tpu as pltpu
from jax.experimental.pallas import tpu_sc as plsc

H = 768
E = 8
FF = 512
SFF = 1024
THR = 0.2
T = 2048

B = 128
CAP = T * 2 + E * B
DUMP = 128
XS_ROWS = CAP + DUMP
NB = XS_ROWS // B

NW = 32
TW = T // NW
CH = 16
LANES = 128


def _router_body(x_ref, wr_ref, w_ref, s_ref, b_ref):
    xb = x_ref[...]
    wr = jnp.concatenate(
        [wr_ref[...], jnp.zeros((H, LANES - E), jnp.float32)], axis=1)
    logits = jnp.dot(xb, wr, preferred_element_type=jnp.float32)
    lane = lax.broadcasted_iota(jnp.int32, (T, LANES), 1)
    valid = lane < E
    lg = jnp.where(valid, logits, -1e30)
    m = jnp.max(lg, axis=1, keepdims=True)
    ex = jnp.where(valid, jnp.exp(lg - m), 0.0)
    z = jnp.sum(ex, axis=1, keepdims=True)
    p = ex / z
    p1 = jnp.max(p, axis=1, keepdims=True)
    i1 = jnp.min(jnp.where(p == p1, lane, LANES - 1), axis=1, keepdims=True)
    pm = jnp.where(lane == i1, -1.0, p)
    p2 = jnp.max(pm, axis=1, keepdims=True)
    i2 = jnp.min(jnp.where((pm == p2) & valid, lane, LANES - 1),
                 axis=1, keepdims=True)
    keep = p1 >= THR
    keepf = keep.astype(jnp.float32)
    w1 = p1 * keepf
    w2 = p2 * keepf
    oh1 = ((lane == i1) & valid).astype(jnp.float32)
    oh2 = ((lane == i2) & valid).astype(jnp.float32)
    oh2s = (lane == i2 + E).astype(jnp.float32)
    ohk = (oh1 + oh2s) * keepf

    C = 256
    r_i = lax.broadcasted_iota(jnp.int32, (C, C), 0)
    c_i = lax.broadcasted_iota(jnp.int32, (C, C), 1)
    tril = (c_i < r_i).astype(jnp.float32)
    carry = jnp.zeros((1, LANES), jnp.float32)
    cums = []
    for ci in range(T // C):
        blk = lax.slice_in_dim(ohk, ci * C, (ci + 1) * C, axis=0)
        cums.append(jnp.dot(tril, blk, preferred_element_type=jnp.float32)
                    + carry)
        carry = carry + jnp.sum(blk, axis=0, keepdims=True)
    cum = jnp.concatenate(cums, axis=0)
    tot = carry

    a_i = lax.broadcasted_iota(jnp.int32, (LANES, LANES), 0)
    e_i = lax.broadcasted_iota(jnp.int32, (LANES, LANES), 1)
    m_count = (((a_i == e_i) | (a_i == e_i + E)) & (e_i < E)).astype(jnp.float32)
    m_prefix = ((a_i < e_i) & (a_i < E) & (e_i < E)).astype(jnp.float32)
    tot8 = jnp.broadcast_to(tot, (8, LANES))
    count8 = jnp.dot(tot8, m_count, preferred_element_type=jnp.float32)
    padded8 = jnp.floor((count8 + (B - 1)) * (1.0 / B)) * B
    po8 = jnp.dot(padded8, m_prefix, preferred_element_type=jnp.float32)
    po = lax.slice_in_dim(po8, 0, 1, axis=0)
    countr = lax.slice_in_dim(count8, 0, 1, axis=0)

    def lsum(mat, oh):
        return jnp.sum(mat * oh, axis=1, keepdims=True)

    po_b = jnp.broadcast_to(po, (T, LANES))
    tot_b = jnp.broadcast_to(tot, (T, LANES))
    slot1 = lsum(po_b, oh1) + lsum(cum, oh1)
    slot2 = lsum(po_b, oh2) + lsum(tot_b, oh2) + lsum(cum, oh2s)
    s1 = slot1.astype(jnp.int32)
    s2 = slot2.astype(jnp.int32)
    tok = lax.broadcasted_iota(jnp.int32, (T, 1), 0)
    ss1 = jnp.where(keep, s1, CAP + (tok * 2) % DUMP)
    ss2 = jnp.where(keep, s2, CAP + (tok * 2 + 1) % DUMP)

    w_ref[...] = jnp.where(lane < 16, w1, jnp.where(lane < 32, w2, 0.0))
    s_ref[...] = jnp.where(lane == 0, ss1, jnp.where(lane == 1, ss2, 0))

    bl = lax.broadcasted_iota(jnp.int32, (NB, LANES), 1)
    brow = lax.broadcasted_iota(jnp.int32, (NB, LANES), 0)
    bstart = (brow * B).astype(jnp.float32)
    po_nb = jnp.broadcast_to(po, (NB, LANES))
    cnt_nb = jnp.broadcast_to(countr, (NB, LANES))
    le = ((po_nb <= bstart) & (bl < E)).astype(jnp.float32)
    be = (jnp.sum(le, axis=1, keepdims=True) - 1.0).astype(jnp.int32)
    ohbe = (bl == be).astype(jnp.float32)
    bend = jnp.sum((po_nb + cnt_nb) * ohbe, axis=1, keepdims=True)
    ba = (lax.slice_in_dim(bstart, 0, 1, axis=1) < bend).astype(jnp.int32)
    b_ref[...] = jnp.where(bl == 0, be, jnp.where(bl == 1, ba, 0))


_router = pl.pallas_call(
    _router_body,
    grid=(1,),
    in_specs=[
        pl.BlockSpec((T, H), lambda i: (0, 0)),
        pl.BlockSpec((H, E), lambda i: (0, 0)),
    ],
    out_specs=[
        pl.BlockSpec((T, LANES), lambda i: (0, 0)),
        pl.BlockSpec((T, LANES), lambda i: (0, 0)),
        pl.BlockSpec((NB, LANES), lambda i: (0, 0)),
    ],
    out_shape=[
        jax.ShapeDtypeStruct((T, LANES), jnp.float32),
        jax.ShapeDtypeStruct((T, LANES), jnp.int32),
        jax.ShapeDtypeStruct((NB, LANES), jnp.int32),
    ],
)


def _ffn_body(be_ref, ba_ref, xs_ref, wg_hbm, wu_hbm, wd_hbm, zs_ref,
              wg_sc, wu_sc, wd_sc, sems):
    b = pl.program_id(0)
    act = ba_ref[b]
    e = be_ref[b]
    elast = jnp.where(b == 0, -1, be_ref[jnp.maximum(b - 1, 0)])

    def _copies(ei):
        return (pltpu.make_async_copy(wg_hbm.at[ei], wg_sc.at[ei],
                                      sems.at[ei]),
                pltpu.make_async_copy(wu_hbm.at[ei], wu_sc.at[ei],
                                      sems.at[ei]),
                pltpu.make_async_copy(wd_hbm.at[ei], wd_sc.at[ei],
                                      sems.at[ei]))

    @pl.when(b == 0)
    def _prefetch():
        for ei in range(E):
            for cp in _copies(ei):
                cp.start()

    for ei in range(E):
        @pl.when((elast < ei) & (ei <= e))
        def _wait(ei=ei):
            for cp in _copies(ei):
                cp.wait()

    @pl.when(act == 1)
    def _compute():
        xb = xs_ref[...].astype(jnp.bfloat16)
        wg = wg_sc[e].astype(jnp.bfloat16)
        wu = wu_sc[e].astype(jnp.bfloat16)
        g = jnp.dot(xb, wg, preferred_element_type=jnp.float32)
        u = jnp.dot(xb, wu, preferred_element_type=jnp.float32)
        h = (g * lax.logistic(g)) * u
        zs_ref[...] = jnp.dot(h.astype(jnp.bfloat16),
                              wd_sc[e].astype(jnp.bfloat16),
                              preferred_element_type=jnp.float32)

    @pl.when(act == 0)
    def _zero():
        zs_ref[...] = jnp.zeros((B, H), jnp.float32)


_grouped_ffn = pl.pallas_call(
    _ffn_body,
    grid_spec=pltpu.PrefetchScalarGridSpec(
        num_scalar_prefetch=2,
        grid=(NB,),
        in_specs=[
            pl.BlockSpec((B, H), lambda b, be, ba: (b, 0)),
            pl.BlockSpec(memory_space=pl.ANY),
            pl.BlockSpec(memory_space=pl.ANY),
            pl.BlockSpec(memory_space=pl.ANY),
        ],
        out_specs=pl.BlockSpec((B, H), lambda b, be, ba: (b, 0)),
        scratch_shapes=[
            pltpu.VMEM((E, H, FF), jnp.float32),
            pltpu.VMEM((E, H, FF), jnp.float32),
            pltpu.VMEM((E, FF, H), jnp.float32),
            pltpu.SemaphoreType.DMA((E,)),
        ],
    ),
    out_shape=jax.ShapeDtypeStruct((XS_ROWS, H), jnp.float32),
)


def _shared_body(x_ref, wg_ref, wu_ref, wd_ref, o_ref):
    xb = x_ref[...].astype(jnp.bfloat16)
    g = jnp.dot(xb, wg_ref[...].astype(jnp.bfloat16),
                preferred_element_type=jnp.float32)
    u = jnp.dot(xb, wu_ref[...].astype(jnp.bfloat16),
                preferred_element_type=jnp.float32)
    h = (g * lax.logistic(g)) * u
    o_ref[...] = jnp.dot(h.astype(jnp.bfloat16),
                         wd_ref[...].astype(jnp.bfloat16),
                         preferred_element_type=jnp.float32)


_shared_ffn = pl.pallas_call(
    _shared_body,
    grid=(2,),
    in_specs=[
        pl.BlockSpec((T // 2, H), lambda i: (i, 0)),
        pl.BlockSpec((H, SFF), lambda i: (0, 0)),
        pl.BlockSpec((H, SFF), lambda i: (0, 0)),
        pl.BlockSpec((SFF, H), lambda i: (0, 0)),
    ],
    out_specs=pl.BlockSpec((T // 2, H), lambda i: (i, 0)),
    out_shape=jax.ShapeDtypeStruct((T, H), jnp.float32),
)


def _extract_columns(sblk_v, col_a, col_b, idx0_v, idx1_v, n):
    lanei = lax.iota(jnp.int32, 16)
    for g in range(n // 16):
        v0 = jnp.zeros((16,), jnp.int32)
        v1 = jnp.zeros((16,), jnp.int32)
        for i in range(16):
            chunk = sblk_v[pl.ds((g * 16 + i) * LANES, 16)]
            v0 = jnp.where(lanei == i, chunk[col_a], v0)
            v1 = jnp.where(lanei == i, chunk[col_b], v1)
        idx0_v[pl.ds(g * 16, 16)] = v0
        idx1_v[pl.ds(g * 16, 16)] = v1


def _sc_dispatch_body(x_hbm, s_hbm, xs_hbm, sblk_v, idx0_v, idx1_v, rows_v,
                      sem0, sem1):
    wid = lax.axis_index("s") * 2 + lax.axis_index("c")
    base = wid * TW
    cpr = pltpu.async_copy(x_hbm.at[pl.ds(base, TW)], rows_v, sem0)
    pltpu.sync_copy(s_hbm.at[pl.ds(base * LANES, TW * LANES)], sblk_v)
    _extract_columns(sblk_v, 0, 1, idx0_v, idx1_v, TW)
    cpr.wait()
    cp0 = pltpu.async_copy(rows_v, xs_hbm.at[idx0_v], sem0)
    cp1 = pltpu.async_copy(rows_v, xs_hbm.at[idx1_v], sem1)
    cp0.wait()
    cp1.wait()


def _sc_combine_body(zs_hbm, sh_hbm, w_hbm, s_hbm, y_hbm,
                     wblk_v, sblk_v, idx0_v, idx1_v, acc_v,
                     zg0a_v, zg1a_v, zg0b_v, zg1b_v,
                     sem_s, sem_a, sem_b, sem_y):
    wid = lax.axis_index("s") * 2 + lax.axis_index("c")
    base = wid * TW
    cps = pltpu.async_copy(sh_hbm.at[pl.ds(base, TW)], acc_v, sem_s)
    pltpu.sync_copy(s_hbm.at[pl.ds(base * LANES, TW * LANES)], sblk_v)
    pltpu.sync_copy(w_hbm.at[pl.ds(base * LANES, TW * LANES)], wblk_v)
    _extract_columns(sblk_v, 0, 1, idx0_v, idx1_v, TW)

    nch = TW // CH
    bufs = [(zg0a_v, zg1a_v, sem_a), (zg0b_v, zg1b_v, sem_b)]

    def start(ch):
        zg0, zg1, sem = bufs[ch % 2]
        d0 = pltpu.async_copy(zs_hbm.at[idx0_v.at[pl.ds(ch * CH, CH)]], zg0,
                              sem)
        d1 = pltpu.async_copy(zs_hbm.at[idx1_v.at[pl.ds(ch * CH, CH)]], zg1,
                              sem)
        return d0, d1

    pending = start(0)
    cps.wait()
    ywaits = []
    for ch in range(nch):
        nxt = start(ch + 1) if ch + 1 < nch else None
        pending[0].wait()
        pending[1].wait()
        zg0, zg1, _ = bufs[ch % 2]

        def row_body(r, carry, ch=ch, zg0=zg0, zg1=zg1):
            rr = ch * CH + r
            a0 = wblk_v[pl.ds(rr * LANES, 16)]
            a1 = wblk_v[pl.ds(rr * LANES + 16, 16)]
            for j in range(H // 16):
                sl = pl.ds(j * 16, 16)
                plsc.addupdate(acc_v.at[rr, sl], zg0[r, sl] * a0
                               + zg1[r, sl] * a1)
            return carry

        lax.fori_loop(0, CH, row_body, 0)
        ywaits.append(pltpu.async_copy(
            acc_v.at[pl.ds(ch * CH, CH)],
            y_hbm.at[pl.ds(base + ch * CH, CH)], sem_y))
        pending = nxt
    for yd in ywaits:
        yd.wait()


@functools.lru_cache(maxsize=1)
def _sc_kernels():
    mesh = plsc.VectorSubcoreMesh(core_axis_name="c", subcore_axis_name="s")
    dispatch = pl.kernel(
        _sc_dispatch_body,
        out_type=jax.ShapeDtypeStruct((XS_ROWS, H), jnp.float32),
        mesh=mesh,
        scratch_types=[
            pltpu.VMEM((TW * LANES,), jnp.int32),
            pltpu.VMEM((TW,), jnp.int32),
            pltpu.VMEM((TW,), jnp.int32),
            pltpu.VMEM((TW, H), jnp.float32),
            pltpu.SemaphoreType.DMA,
            pltpu.SemaphoreType.DMA,
        ],
    )
    combine = pl.kernel(
        _sc_combine_body,
        out_type=jax.ShapeDtypeStruct((T, H), jnp.float32),
        mesh=mesh,
        scratch_types=[
            pltpu.VMEM((TW * LANES,), jnp.float32),
            pltpu.VMEM((TW * LANES,), jnp.int32),
            pltpu.VMEM((TW,), jnp.int32),
            pltpu.VMEM((TW,), jnp.int32),
            pltpu.VMEM((TW, H), jnp.float32),
            pltpu.VMEM((CH, H), jnp.float32),
            pltpu.VMEM((CH, H), jnp.float32),
            pltpu.VMEM((CH, H), jnp.float32),
            pltpu.VMEM((CH, H), jnp.float32),
            pltpu.SemaphoreType.DMA,
            pltpu.SemaphoreType.DMA,
            pltpu.SemaphoreType.DMA,
            pltpu.SemaphoreType.DMA,
        ],
    )
    return dispatch, combine


def kernel(hidden_states, Wr, Wg, Wu, Wd, Wg_s, Wu_s, Wd_s):
    x = hidden_states
    wout, sout, bout = _router(x, Wr)
    be = bout[:, 0]
    ba = bout[:, 1]
    sc_dispatch, sc_combine = _sc_kernels()
    sflat = sout.reshape(-1)
    xs = sc_dispatch(x, sflat)
    zs = _grouped_ffn(be, ba, xs, Wg, Wu, Wd)
    sh = _shared_ffn(x, Wg_s, Wu_s, Wd_s)
    return sc_combine(zs, sh, wout.reshape(-1), sflat)


# speedup vs baseline: 1.0703x; 1.0703x over previous
import functools

import jax
import jax.numpy as jnp
from jax import lax
from jax.experimental import pallas as pl
from jax.experimental.pallas import tpu as pltpu
from jax.experimental.pallas import tpu_sc as plsc

H = 768
E = 8
FF = 512
SFF = 1024
THR = 0.2
T = 2048

B = 256
CAP = T * 2 + E * B
DUMP = 256
XS_ROWS = CAP + DUMP
NB = XS_ROWS // B

NW = 32
TW = T // NW
CH = 16
LANES = 128


def _router_body(x_ref, wr_ref, w_ref, s_ref, b_ref):
    xb = x_ref[...]
    wr = jnp.concatenate(
        [wr_ref[...], jnp.zeros((H, LANES - E), jnp.float32)], axis=1)
    logits = jnp.dot(xb, wr, preferred_element_type=jnp.float32)
    lane = lax.broadcasted_iota(jnp.int32, (T, LANES), 1)
    valid = lane < E
    lg = jnp.where(valid, logits, -1e30)
    m = jnp.max(lg, axis=1, keepdims=True)
    ex = jnp.where(valid, jnp.exp(lg - m), 0.0)
    z = jnp.sum(ex, axis=1, keepdims=True)
    p = ex / z
    p1 = jnp.max(p, axis=1, keepdims=True)
    i1 = jnp.min(jnp.where(p == p1, lane, LANES - 1), axis=1, keepdims=True)
    pm = jnp.where(lane == i1, -1.0, p)
    p2 = jnp.max(pm, axis=1, keepdims=True)
    i2 = jnp.min(jnp.where((pm == p2) & valid, lane, LANES - 1),
                 axis=1, keepdims=True)
    keep = p1 >= THR
    keepf = keep.astype(jnp.float32)
    w1 = p1 * keepf
    w2 = p2 * keepf
    oh1 = ((lane == i1) & valid).astype(jnp.float32)
    oh2 = ((lane == i2) & valid).astype(jnp.float32)
    oh2s = (lane == i2 + E).astype(jnp.float32)
    ohk = (oh1 + oh2s) * keepf

    C = 256
    r_i = lax.broadcasted_iota(jnp.int32, (C, C), 0)
    c_i = lax.broadcasted_iota(jnp.int32, (C, C), 1)
    tril = (c_i < r_i).astype(jnp.float32)
    carry = jnp.zeros((1, LANES), jnp.float32)
    cums = []
    for ci in range(T // C):
        blk = lax.slice_in_dim(ohk, ci * C, (ci + 1) * C, axis=0)
        cums.append(jnp.dot(tril, blk, preferred_element_type=jnp.float32)
                    + carry)
        carry = carry + jnp.sum(blk, axis=0, keepdims=True)
    cum = jnp.concatenate(cums, axis=0)
    tot = carry

    a_i = lax.broadcasted_iota(jnp.int32, (LANES, LANES), 0)
    e_i = lax.broadcasted_iota(jnp.int32, (LANES, LANES), 1)
    m_count = (((a_i == e_i) | (a_i == e_i + E)) & (e_i < E)).astype(jnp.float32)
    m_prefix = ((a_i < e_i) & (a_i < E) & (e_i < E)).astype(jnp.float32)
    tot8 = jnp.broadcast_to(tot, (8, LANES))
    count8 = jnp.dot(tot8, m_count, preferred_element_type=jnp.float32)
    padded8 = jnp.floor((count8 + (B - 1)) * (1.0 / B)) * B
    po8 = jnp.dot(padded8, m_prefix, preferred_element_type=jnp.float32)
    po = lax.slice_in_dim(po8, 0, 1, axis=0)
    countr = lax.slice_in_dim(count8, 0, 1, axis=0)

    def lsum(mat, oh):
        return jnp.sum(mat * oh, axis=1, keepdims=True)

    po_b = jnp.broadcast_to(po, (T, LANES))
    tot_b = jnp.broadcast_to(tot, (T, LANES))
    slot1 = lsum(po_b, oh1) + lsum(cum, oh1)
    slot2 = lsum(po_b, oh2) + lsum(tot_b, oh2) + lsum(cum, oh2s)
    s1 = slot1.astype(jnp.int32)
    s2 = slot2.astype(jnp.int32)
    tok = lax.broadcasted_iota(jnp.int32, (T, 1), 0)
    ss1 = jnp.where(keep, s1, CAP + (tok * 2) % DUMP)
    ss2 = jnp.where(keep, s2, CAP + (tok * 2 + 1) % DUMP)

    w_ref[...] = jnp.where(lane < 16, w1, jnp.where(lane < 32, w2, 0.0))
    s_ref[...] = jnp.where(lane == 0, ss1, jnp.where(lane == 1, ss2, 0))

    bl = lax.broadcasted_iota(jnp.int32, (NB, LANES), 1)
    brow = lax.broadcasted_iota(jnp.int32, (NB, LANES), 0)
    bstart = (brow * B).astype(jnp.float32)
    po_nb = jnp.broadcast_to(po, (NB, LANES))
    cnt_nb = jnp.broadcast_to(countr, (NB, LANES))
    le = ((po_nb <= bstart) & (bl < E)).astype(jnp.float32)
    be = (jnp.sum(le, axis=1, keepdims=True) - 1.0).astype(jnp.int32)
    ohbe = (bl == be).astype(jnp.float32)
    bend = jnp.sum((po_nb + cnt_nb) * ohbe, axis=1, keepdims=True)
    ba = (lax.slice_in_dim(bstart, 0, 1, axis=1) < bend).astype(jnp.int32)
    b_ref[...] = jnp.where(bl == 0, be, jnp.where(bl == 1, ba, 0))


_router = pl.pallas_call(
    _router_body,
    grid=(1,),
    in_specs=[
        pl.BlockSpec((T, H), lambda i: (0, 0)),
        pl.BlockSpec((H, E), lambda i: (0, 0)),
    ],
    out_specs=[
        pl.BlockSpec((T, LANES), lambda i: (0, 0)),
        pl.BlockSpec((T, LANES), lambda i: (0, 0)),
        pl.BlockSpec((NB, LANES), lambda i: (0, 0)),
    ],
    out_shape=[
        jax.ShapeDtypeStruct((T, LANES), jnp.float32),
        jax.ShapeDtypeStruct((T, LANES), jnp.int32),
        jax.ShapeDtypeStruct((NB, LANES), jnp.int32),
    ],
)


def _ffn_body(be_ref, ba_ref, xs_ref, wg_hbm, wu_hbm, wd_hbm, zs_ref,
              wg_sc, wu_sc, wd_sc, sems):
    b = pl.program_id(0)
    act = ba_ref[b]
    e = be_ref[b]
    elast = jnp.where(b == 0, -1, be_ref[jnp.maximum(b - 1, 0)])

    def _copies(ei):
        return (pltpu.make_async_copy(wg_hbm.at[ei], wg_sc.at[ei],
                                      sems.at[ei]),
                pltpu.make_async_copy(wu_hbm.at[ei], wu_sc.at[ei],
                                      sems.at[ei]),
                pltpu.make_async_copy(wd_hbm.at[ei], wd_sc.at[ei],
                                      sems.at[ei]))

    @pl.when(b == 0)
    def _prefetch():
        for ei in range(E):
            for cp in _copies(ei):
                cp.start()

    for ei in range(E):
        @pl.when((elast < ei) & (ei <= e))
        def _wait(ei=ei):
            for cp in _copies(ei):
                cp.wait()

    @pl.when(act == 1)
    def _compute():
        xb = xs_ref[...].astype(jnp.bfloat16)
        wg = wg_sc[e].astype(jnp.bfloat16)
        wu = wu_sc[e].astype(jnp.bfloat16)
        g = jnp.dot(xb, wg, preferred_element_type=jnp.float32)
        u = jnp.dot(xb, wu, preferred_element_type=jnp.float32)
        h = (g * lax.logistic(g)) * u
        zs_ref[...] = jnp.dot(h.astype(jnp.bfloat16),
                              wd_sc[e].astype(jnp.bfloat16),
                              preferred_element_type=jnp.float32)

    @pl.when(act == 0)
    def _zero():
        zs_ref[...] = jnp.zeros((B, H), jnp.float32)


_grouped_ffn = pl.pallas_call(
    _ffn_body,
    grid_spec=pltpu.PrefetchScalarGridSpec(
        num_scalar_prefetch=2,
        grid=(NB,),
        in_specs=[
            pl.BlockSpec((B, H), lambda b, be, ba: (b, 0)),
            pl.BlockSpec(memory_space=pl.ANY),
            pl.BlockSpec(memory_space=pl.ANY),
            pl.BlockSpec(memory_space=pl.ANY),
        ],
        out_specs=pl.BlockSpec((B, H), lambda b, be, ba: (b, 0)),
        scratch_shapes=[
            pltpu.VMEM((E, H, FF), jnp.float32),
            pltpu.VMEM((E, H, FF), jnp.float32),
            pltpu.VMEM((E, FF, H), jnp.float32),
            pltpu.SemaphoreType.DMA((E,)),
        ],
    ),
    out_shape=jax.ShapeDtypeStruct((XS_ROWS, H), jnp.float32),
)


def _shared_body(x_ref, wg_ref, wu_ref, wd_ref, o_ref):
    xb = x_ref[...].astype(jnp.bfloat16)
    g = jnp.dot(xb, wg_ref[...].astype(jnp.bfloat16),
                preferred_element_type=jnp.float32)
    u = jnp.dot(xb, wu_ref[...].astype(jnp.bfloat16),
                preferred_element_type=jnp.float32)
    h = (g * lax.logistic(g)) * u
    o_ref[...] = jnp.dot(h.astype(jnp.bfloat16),
                         wd_ref[...].astype(jnp.bfloat16),
                         preferred_element_type=jnp.float32)


_shared_ffn = pl.pallas_call(
    _shared_body,
    grid=(2,),
    in_specs=[
        pl.BlockSpec((T // 2, H), lambda i: (i, 0)),
        pl.BlockSpec((H, SFF), lambda i: (0, 0)),
        pl.BlockSpec((H, SFF), lambda i: (0, 0)),
        pl.BlockSpec((SFF, H), lambda i: (0, 0)),
    ],
    out_specs=pl.BlockSpec((T // 2, H), lambda i: (i, 0)),
    out_shape=jax.ShapeDtypeStruct((T, H), jnp.float32),
)


def _extract_columns(sblk_v, col_a, col_b, idx0_v, idx1_v, n):
    lanei = lax.iota(jnp.int32, 16)
    for g in range(n // 16):
        v0 = jnp.zeros((16,), jnp.int32)
        v1 = jnp.zeros((16,), jnp.int32)
        for i in range(16):
            chunk = sblk_v[pl.ds((g * 16 + i) * LANES, 16)]
            v0 = jnp.where(lanei == i, chunk[col_a], v0)
            v1 = jnp.where(lanei == i, chunk[col_b], v1)
        idx0_v[pl.ds(g * 16, 16)] = v0
        idx1_v[pl.ds(g * 16, 16)] = v1


def _sc_dispatch_body(x_hbm, s_hbm, xs_hbm, sblk_v, idx0_v, idx1_v, rows_v,
                      sem0, sem1):
    wid = lax.axis_index("s") * 2 + lax.axis_index("c")
    base = wid * TW
    cpr = pltpu.async_copy(x_hbm.at[pl.ds(base, TW)], rows_v, sem0)
    pltpu.sync_copy(s_hbm.at[pl.ds(base * LANES, TW * LANES)], sblk_v)
    _extract_columns(sblk_v, 0, 1, idx0_v, idx1_v, TW)
    cpr.wait()
    cp0 = pltpu.async_copy(rows_v, xs_hbm.at[idx0_v], sem0)
    cp1 = pltpu.async_copy(rows_v, xs_hbm.at[idx1_v], sem1)
    cp0.wait()
    cp1.wait()


def _sc_combine_body(zs_hbm, sh_hbm, w_hbm, s_hbm, y_hbm,
                     wblk_v, sblk_v, idx0_v, idx1_v, acc_v,
                     zg0a_v, zg1a_v, zg0b_v, zg1b_v,
                     sem_s, sem_a, sem_b, sem_y):
    wid = lax.axis_index("s") * 2 + lax.axis_index("c")
    base = wid * TW
    cps = pltpu.async_copy(sh_hbm.at[pl.ds(base, TW)], acc_v, sem_s)
    pltpu.sync_copy(s_hbm.at[pl.ds(base * LANES, TW * LANES)], sblk_v)
    pltpu.sync_copy(w_hbm.at[pl.ds(base * LANES, TW * LANES)], wblk_v)
    _extract_columns(sblk_v, 0, 1, idx0_v, idx1_v, TW)

    nch = TW // CH
    bufs = [(zg0a_v, zg1a_v, sem_a), (zg0b_v, zg1b_v, sem_b)]

    def start(ch):
        zg0, zg1, sem = bufs[ch % 2]
        d0 = pltpu.async_copy(zs_hbm.at[idx0_v.at[pl.ds(ch * CH, CH)]], zg0,
                              sem)
        d1 = pltpu.async_copy(zs_hbm.at[idx1_v.at[pl.ds(ch * CH, CH)]], zg1,
                              sem)
        return d0, d1

    pending = start(0)
    cps.wait()
    ywaits = []
    for ch in range(nch):
        nxt = start(ch + 1) if ch + 1 < nch else None
        pending[0].wait()
        pending[1].wait()
        zg0, zg1, _ = bufs[ch % 2]

        def row_body(r, carry, ch=ch, zg0=zg0, zg1=zg1):
            rr = ch * CH + r
            a0 = wblk_v[pl.ds(rr * LANES, 16)]
            a1 = wblk_v[pl.ds(rr * LANES + 16, 16)]
            for j in range(H // 16):
                sl = pl.ds(j * 16, 16)
                plsc.addupdate(acc_v.at[rr, sl], zg0[r, sl] * a0
                               + zg1[r, sl] * a1)
            return carry

        lax.fori_loop(0, CH, row_body, 0)
        ywaits.append(pltpu.async_copy(
            acc_v.at[pl.ds(ch * CH, CH)],
            y_hbm.at[pl.ds(base + ch * CH, CH)], sem_y))
        pending = nxt
    for yd in ywaits:
        yd.wait()


@functools.lru_cache(maxsize=1)
def _sc_kernels():
    mesh = plsc.VectorSubcoreMesh(core_axis_name="c", subcore_axis_name="s")
    dispatch = pl.kernel(
        _sc_dispatch_body,
        out_type=jax.ShapeDtypeStruct((XS_ROWS, H), jnp.float32),
        mesh=mesh,
        scratch_types=[
            pltpu.VMEM((TW * LANES,), jnp.int32),
            pltpu.VMEM((TW,), jnp.int32),
            pltpu.VMEM((TW,), jnp.int32),
            pltpu.VMEM((TW, H), jnp.float32),
            pltpu.SemaphoreType.DMA,
            pltpu.SemaphoreType.DMA,
        ],
    )
    combine = pl.kernel(
        _sc_combine_body,
        out_type=jax.ShapeDtypeStruct((T, H), jnp.float32),
        mesh=mesh,
        scratch_types=[
            pltpu.VMEM((TW * LANES,), jnp.float32),
            pltpu.VMEM((TW * LANES,), jnp.int32),
            pltpu.VMEM((TW,), jnp.int32),
            pltpu.VMEM((TW,), jnp.int32),
            pltpu.VMEM((TW, H), jnp.float32),
            pltpu.VMEM((CH, H), jnp.float32),
            pltpu.VMEM((CH, H), jnp.float32),
            pltpu.VMEM((CH, H), jnp.float32),
            pltpu.VMEM((CH, H), jnp.float32),
            pltpu.SemaphoreType.DMA,
            pltpu.SemaphoreType.DMA,
            pltpu.SemaphoreType.DMA,
            pltpu.SemaphoreType.DMA,
        ],
    )
    return dispatch, combine


def kernel(hidden_states, Wr, Wg, Wu, Wd, Wg_s, Wu_s, Wd_s):
    x = hidden_states
    wout, sout, bout = _router(x, Wr)
    be = bout[:, 0]
    ba = bout[:, 1]
    sc_dispatch, sc_combine = _sc_kernels()
    sflat = sout.reshape(-1)
    xs = sc_dispatch(x, sflat)
    zs = _grouped_ffn(be, ba, xs, Wg, Wu, Wd)
    sh = _shared_ffn(x, Wg_s, Wu_s, Wd_s)
    return sc_combine(zs, sh, wout.reshape(-1), sflat)


# speedup vs baseline: 1.1260x; 1.0520x over previous
import functools

import jax
import jax.numpy as jnp
from jax import lax
from jax.experimental import pallas as pl
from jax.experimental.pallas import tpu as pltpu
from jax.experimental.pallas import tpu_sc as plsc

H = 768
E = 8
FF = 512
SFF = 1024
THR = 0.2
T = 2048

B = 256
CAP = T * 2 + E * B
DUMP = 256
XS_ROWS = CAP + DUMP
NB = XS_ROWS // B

NW = 32
TW = T // NW
CH = 16
LANES = 128


def _router_body(x_ref, wr_ref, w_ref, s_ref, b_ref):
    xb = x_ref[...]
    wr = jnp.concatenate(
        [wr_ref[...], jnp.zeros((H, LANES - E), jnp.float32)], axis=1)
    logits = jnp.dot(xb, wr, preferred_element_type=jnp.float32)
    lane = lax.broadcasted_iota(jnp.int32, (T, LANES), 1)
    valid = lane < E
    lg = jnp.where(valid, logits, -1e30)
    m = jnp.max(lg, axis=1, keepdims=True)
    ex = jnp.where(valid, jnp.exp(lg - m), 0.0)
    z = jnp.sum(ex, axis=1, keepdims=True)
    p = ex / z
    p1 = jnp.max(p, axis=1, keepdims=True)
    i1 = jnp.min(jnp.where(p == p1, lane, LANES - 1), axis=1, keepdims=True)
    pm = jnp.where(lane == i1, -1.0, p)
    p2 = jnp.max(pm, axis=1, keepdims=True)
    i2 = jnp.min(jnp.where((pm == p2) & valid, lane, LANES - 1),
                 axis=1, keepdims=True)
    keep = p1 >= THR
    keepf = keep.astype(jnp.float32)
    w1 = p1 * keepf
    w2 = p2 * keepf
    oh1 = ((lane == i1) & valid).astype(jnp.float32)
    oh2 = ((lane == i2) & valid).astype(jnp.float32)
    oh2s = (lane == i2 + E).astype(jnp.float32)
    ohk = (oh1 + oh2s) * keepf

    C = 256
    r_i = lax.broadcasted_iota(jnp.int32, (C, C), 0)
    c_i = lax.broadcasted_iota(jnp.int32, (C, C), 1)
    tril = (c_i < r_i).astype(jnp.float32)
    carry = jnp.zeros((1, LANES), jnp.float32)
    cums = []
    for ci in range(T // C):
        blk = lax.slice_in_dim(ohk, ci * C, (ci + 1) * C, axis=0)
        cums.append(jnp.dot(tril, blk, preferred_element_type=jnp.float32)
                    + carry)
        carry = carry + jnp.sum(blk, axis=0, keepdims=True)
    cum = jnp.concatenate(cums, axis=0)
    tot = carry

    a_i = lax.broadcasted_iota(jnp.int32, (LANES, LANES), 0)
    e_i = lax.broadcasted_iota(jnp.int32, (LANES, LANES), 1)
    m_count = (((a_i == e_i) | (a_i == e_i + E)) & (e_i < E)).astype(jnp.float32)
    m_prefix = ((a_i < e_i) & (a_i < E) & (e_i < E)).astype(jnp.float32)
    tot8 = jnp.broadcast_to(tot, (8, LANES))
    count8 = jnp.dot(tot8, m_count, preferred_element_type=jnp.float32)
    padded8 = jnp.floor((count8 + (B - 1)) * (1.0 / B)) * B
    po8 = jnp.dot(padded8, m_prefix, preferred_element_type=jnp.float32)
    po = lax.slice_in_dim(po8, 0, 1, axis=0)
    countr = lax.slice_in_dim(count8, 0, 1, axis=0)

    def lsum(mat, oh):
        return jnp.sum(mat * oh, axis=1, keepdims=True)

    po_b = jnp.broadcast_to(po, (T, LANES))
    tot_b = jnp.broadcast_to(tot, (T, LANES))
    slot1 = lsum(po_b, oh1) + lsum(cum, oh1)
    slot2 = lsum(po_b, oh2) + lsum(tot_b, oh2) + lsum(cum, oh2s)
    s1 = slot1.astype(jnp.int32)
    s2 = slot2.astype(jnp.int32)
    tok = lax.broadcasted_iota(jnp.int32, (T, 1), 0)
    ss1 = jnp.where(keep, s1, CAP + (tok * 2) % DUMP)
    ss2 = jnp.where(keep, s2, CAP + (tok * 2 + 1) % DUMP)

    w_ref[...] = jnp.where(lane < 16, w1, jnp.where(lane < 32, w2, 0.0))
    s_ref[...] = jnp.where(lane == 0, ss1, jnp.where(lane == 1, ss2, 0))

    bl = lax.broadcasted_iota(jnp.int32, (NB, LANES), 1)
    brow = lax.broadcasted_iota(jnp.int32, (NB, LANES), 0)
    bstart = (brow * B).astype(jnp.float32)
    po_nb = jnp.broadcast_to(po, (NB, LANES))
    cnt_nb = jnp.broadcast_to(countr, (NB, LANES))
    le = ((po_nb <= bstart) & (bl < E)).astype(jnp.float32)
    be = (jnp.sum(le, axis=1, keepdims=True) - 1.0).astype(jnp.int32)
    ohbe = (bl == be).astype(jnp.float32)
    bend = jnp.sum((po_nb + cnt_nb) * ohbe, axis=1, keepdims=True)
    ba = (lax.slice_in_dim(bstart, 0, 1, axis=1) < bend).astype(jnp.int32)
    b_ref[...] = jnp.where(bl == 0, be, jnp.where(bl == 1, ba, 0))


_router = pl.pallas_call(
    _router_body,
    grid=(1,),
    in_specs=[
        pl.BlockSpec((T, H), lambda i: (0, 0)),
        pl.BlockSpec((H, E), lambda i: (0, 0)),
    ],
    out_specs=[
        pl.BlockSpec((T, LANES), lambda i: (0, 0)),
        pl.BlockSpec((T, LANES), lambda i: (0, 0)),
        pl.BlockSpec((NB, LANES), lambda i: (0, 0)),
    ],
    out_shape=[
        jax.ShapeDtypeStruct((T, LANES), jnp.float32),
        jax.ShapeDtypeStruct((T, LANES), jnp.int32),
        jax.ShapeDtypeStruct((NB, LANES), jnp.int32),
    ],
)


RING = 3


def _ffn_body(be_ref, ba_ref, xs_hbm, wg_hbm, wu_hbm, wd_hbm, zs_hbm,
              wg_sc, wu_sc, wd_sc, ring, zbuf, zero_buf,
              wsems, lsems, osems):
    def wcp(ei):
        return (pltpu.make_async_copy(wg_hbm.at[ei], wg_sc.at[ei],
                                      wsems.at[ei]),
                pltpu.make_async_copy(wu_hbm.at[ei], wu_sc.at[ei],
                                      wsems.at[ei]),
                pltpu.make_async_copy(wd_hbm.at[ei], wd_sc.at[ei],
                                      wsems.at[ei]))

    for ei in range(E):
        for cp in wcp(ei):
            cp.start()
    zero_buf[...] = jnp.zeros((B, H), jnp.float32)

    def ld(blk, slot):
        return pltpu.make_async_copy(xs_hbm.at[pl.ds(blk * B, B)],
                                     ring.at[slot], lsems.at[slot])

    ld(0, 0).start()
    ld(1, 1).start()

    def loop(blk, elast):
        slot = lax.rem(blk, RING)
        oslot = lax.rem(blk, 2)
        nblk = blk + 2

        @pl.when(nblk < NB)
        def _():
            ld(nblk, lax.rem(nblk, RING)).start()

        ld(blk, slot).wait()
        e = be_ref[blk]
        for ei in range(E):
            @pl.when((elast < ei) & (ei <= e))
            def _(ei=ei):
                for cp in wcp(ei):
                    cp.wait()

        @pl.when(blk >= 2)
        def _():
            pltpu.make_async_copy(zbuf.at[0], zs_hbm.at[pl.ds(0, B)],
                                  osems.at[oslot]).wait()

        act = ba_ref[blk]

        @pl.when(act == 1)
        def _compute():
            xb = ring[slot].astype(jnp.bfloat16)
            wg = wg_sc[e].astype(jnp.bfloat16)
            wu = wu_sc[e].astype(jnp.bfloat16)
            g = jnp.dot(xb, wg, preferred_element_type=jnp.float32)
            u = jnp.dot(xb, wu, preferred_element_type=jnp.float32)
            h = (g * lax.logistic(g)) * u
            zbuf[oslot] = jnp.dot(h.astype(jnp.bfloat16),
                                  wd_sc[e].astype(jnp.bfloat16),
                                  preferred_element_type=jnp.float32)
            pltpu.make_async_copy(zbuf.at[oslot],
                                  zs_hbm.at[pl.ds(blk * B, B)],
                                  osems.at[oslot]).start()

        @pl.when(act == 0)
        def _zero():
            pltpu.make_async_copy(zero_buf, zs_hbm.at[pl.ds(blk * B, B)],
                                  osems.at[oslot]).start()

        return e

    lax.fori_loop(0, NB, loop, jnp.int32(-1))
    pltpu.make_async_copy(zbuf.at[0], zs_hbm.at[pl.ds(0, B)],
                          osems.at[0]).wait()
    pltpu.make_async_copy(zbuf.at[0], zs_hbm.at[pl.ds(0, B)],
                          osems.at[1]).wait()


_grouped_ffn = pl.pallas_call(
    _ffn_body,
    grid_spec=pltpu.PrefetchScalarGridSpec(
        num_scalar_prefetch=2,
        grid=(1,),
        in_specs=[
            pl.BlockSpec(memory_space=pl.ANY),
            pl.BlockSpec(memory_space=pl.ANY),
            pl.BlockSpec(memory_space=pl.ANY),
            pl.BlockSpec(memory_space=pl.ANY),
        ],
        out_specs=pl.BlockSpec(memory_space=pl.ANY),
        scratch_shapes=[
            pltpu.VMEM((E, H, FF), jnp.float32),
            pltpu.VMEM((E, H, FF), jnp.float32),
            pltpu.VMEM((E, FF, H), jnp.float32),
            pltpu.VMEM((RING, B, H), jnp.float32),
            pltpu.VMEM((2, B, H), jnp.float32),
            pltpu.VMEM((B, H), jnp.float32),
            pltpu.SemaphoreType.DMA((E,)),
            pltpu.SemaphoreType.DMA((RING,)),
            pltpu.SemaphoreType.DMA((2,)),
        ],
    ),
    out_shape=jax.ShapeDtypeStruct((XS_ROWS, H), jnp.float32),
)


def _shared_body(x_ref, wg_ref, wu_ref, wd_ref, o_ref):
    xb = x_ref[...].astype(jnp.bfloat16)
    g = jnp.dot(xb, wg_ref[...].astype(jnp.bfloat16),
                preferred_element_type=jnp.float32)
    u = jnp.dot(xb, wu_ref[...].astype(jnp.bfloat16),
                preferred_element_type=jnp.float32)
    h = (g * lax.logistic(g)) * u
    o_ref[...] = jnp.dot(h.astype(jnp.bfloat16),
                         wd_ref[...].astype(jnp.bfloat16),
                         preferred_element_type=jnp.float32)


_shared_ffn = pl.pallas_call(
    _shared_body,
    grid=(2,),
    in_specs=[
        pl.BlockSpec((T // 2, H), lambda i: (i, 0)),
        pl.BlockSpec((H, SFF), lambda i: (0, 0)),
        pl.BlockSpec((H, SFF), lambda i: (0, 0)),
        pl.BlockSpec((SFF, H), lambda i: (0, 0)),
    ],
    out_specs=pl.BlockSpec((T // 2, H), lambda i: (i, 0)),
    out_shape=jax.ShapeDtypeStruct((T, H), jnp.float32),
)


def _extract_columns(sblk_v, col_a, col_b, idx0_v, idx1_v, n):
    lanei = lax.iota(jnp.int32, 16)
    for g in range(n // 16):
        v0 = jnp.zeros((16,), jnp.int32)
        v1 = jnp.zeros((16,), jnp.int32)
        for i in range(16):
            chunk = sblk_v[pl.ds((g * 16 + i) * LANES, 16)]
            v0 = jnp.where(lanei == i, chunk[col_a], v0)
            v1 = jnp.where(lanei == i, chunk[col_b], v1)
        idx0_v[pl.ds(g * 16, 16)] = v0
        idx1_v[pl.ds(g * 16, 16)] = v1


def _sc_dispatch_body(x_hbm, s_hbm, xs_hbm, sblk_v, idx0_v, idx1_v, rows_v,
                      sem0, sem1):
    wid = lax.axis_index("s") * 2 + lax.axis_index("c")
    base = wid * TW
    cpr = pltpu.async_copy(x_hbm.at[pl.ds(base, TW)], rows_v, sem0)
    pltpu.sync_copy(s_hbm.at[pl.ds(base * LANES, TW * LANES)], sblk_v)
    _extract_columns(sblk_v, 0, 1, idx0_v, idx1_v, TW)
    cpr.wait()
    cp0 = pltpu.async_copy(rows_v, xs_hbm.at[idx0_v], sem0)
    cp1 = pltpu.async_copy(rows_v, xs_hbm.at[idx1_v], sem1)
    cp0.wait()
    cp1.wait()


def _sc_combine_body(zs_hbm, sh_hbm, w_hbm, s_hbm, y_hbm,
                     wblk_v, sblk_v, idx0_v, idx1_v, acc_v,
                     zg0a_v, zg1a_v, zg0b_v, zg1b_v,
                     sem_s, sem_a, sem_b, sem_y):
    wid = lax.axis_index("s") * 2 + lax.axis_index("c")
    base = wid * TW
    cps = pltpu.async_copy(sh_hbm.at[pl.ds(base, TW)], acc_v, sem_s)
    pltpu.sync_copy(s_hbm.at[pl.ds(base * LANES, TW * LANES)], sblk_v)
    pltpu.sync_copy(w_hbm.at[pl.ds(base * LANES, TW * LANES)], wblk_v)
    _extract_columns(sblk_v, 0, 1, idx0_v, idx1_v, TW)

    nch = TW // CH
    bufs = [(zg0a_v, zg1a_v, sem_a), (zg0b_v, zg1b_v, sem_b)]

    def start(ch):
        zg0, zg1, sem = bufs[ch % 2]
        d0 = pltpu.async_copy(zs_hbm.at[idx0_v.at[pl.ds(ch * CH, CH)]], zg0,
                              sem)
        d1 = pltpu.async_copy(zs_hbm.at[idx1_v.at[pl.ds(ch * CH, CH)]], zg1,
                              sem)
        return d0, d1

    pending = start(0)
    cps.wait()
    ywaits = []
    for ch in range(nch):
        nxt = start(ch + 1) if ch + 1 < nch else None
        pending[0].wait()
        pending[1].wait()
        zg0, zg1, _ = bufs[ch % 2]

        def row_body(r, carry, ch=ch, zg0=zg0, zg1=zg1):
            rr = ch * CH + r
            a0 = wblk_v[pl.ds(rr * LANES, 16)]
            a1 = wblk_v[pl.ds(rr * LANES + 16, 16)]
            for j in range(H // 16):
                sl = pl.ds(j * 16, 16)
                plsc.addupdate(acc_v.at[rr, sl], zg0[r, sl] * a0
                               + zg1[r, sl] * a1)
            return carry

        lax.fori_loop(0, CH, row_body, 0)
        ywaits.append(pltpu.async_copy(
            acc_v.at[pl.ds(ch * CH, CH)],
            y_hbm.at[pl.ds(base + ch * CH, CH)], sem_y))
        pending = nxt
    for yd in ywaits:
        yd.wait()


@functools.lru_cache(maxsize=1)
def _sc_kernels():
    mesh = plsc.VectorSubcoreMesh(core_axis_name="c", subcore_axis_name="s")
    dispatch = pl.kernel(
        _sc_dispatch_body,
        out_type=jax.ShapeDtypeStruct((XS_ROWS, H), jnp.float32),
        mesh=mesh,
        scratch_types=[
            pltpu.VMEM((TW * LANES,), jnp.int32),
            pltpu.VMEM((TW,), jnp.int32),
            pltpu.VMEM((TW,), jnp.int32),
            pltpu.VMEM((TW, H), jnp.float32),
            pltpu.SemaphoreType.DMA,
            pltpu.SemaphoreType.DMA,
        ],
    )
    combine = pl.kernel(
        _sc_combine_body,
        out_type=jax.ShapeDtypeStruct((T, H), jnp.float32),
        mesh=mesh,
        scratch_types=[
            pltpu.VMEM((TW * LANES,), jnp.float32),
            pltpu.VMEM((TW * LANES,), jnp.int32),
            pltpu.VMEM((TW,), jnp.int32),
            pltpu.VMEM((TW,), jnp.int32),
            pltpu.VMEM((TW, H), jnp.float32),
            pltpu.VMEM((CH, H), jnp.float32),
            pltpu.VMEM((CH, H), jnp.float32),
            pltpu.VMEM((CH, H), jnp.float32),
            pltpu.VMEM((CH, H), jnp.float32),
            pltpu.SemaphoreType.DMA,
            pltpu.SemaphoreType.DMA,
            pltpu.SemaphoreType.DMA,
            pltpu.SemaphoreType.DMA,
        ],
    )
    return dispatch, combine


def kernel(hidden_states, Wr, Wg, Wu, Wd, Wg_s, Wu_s, Wd_s):
    x = hidden_states
    wout, sout, bout = _router(x, Wr)
    be = bout[:, 0]
    ba = bout[:, 1]
    sc_dispatch, sc_combine = _sc_kernels()
    sflat = sout.reshape(-1)
    xs = sc_dispatch(x, sflat)
    zs = _grouped_ffn(be, ba, xs, Wg, Wu, Wd)
    sh = _shared_ffn(x, Wg_s, Wu_s, Wd_s)
    return sc_combine(zs, sh, wout.reshape(-1), sflat)


# speedup vs baseline: 1.2194x; 1.0830x over previous
import functools

import jax
import jax.numpy as jnp
from jax import lax
from jax.experimental import pallas as pl
from jax.experimental.pallas import tpu as pltpu
from jax.experimental.pallas import tpu_sc as plsc

H = 768
E = 8
FF = 512
SFF = 1024
THR = 0.2
T = 2048

B = 256
CAP = T * 2 + E * B
DUMP = 256
XS_ROWS = CAP + DUMP
NB = XS_ROWS // B

NW = 32
TW = T // NW
CH = 16
LANES = 128


def _router_body(x_ref, wr_ref, w_ref, s_ref, b_ref):
    xb = x_ref[...]
    wr = jnp.concatenate(
        [wr_ref[...], jnp.zeros((H, LANES - E), jnp.float32)], axis=1)
    logits = jnp.dot(xb, wr, preferred_element_type=jnp.float32)
    lane = lax.broadcasted_iota(jnp.int32, (T, LANES), 1)
    valid = lane < E
    lg = jnp.where(valid, logits, -1e30)
    m = jnp.max(lg, axis=1, keepdims=True)
    ex = jnp.where(valid, jnp.exp(lg - m), 0.0)
    z = jnp.sum(ex, axis=1, keepdims=True)
    p = ex / z
    p1 = jnp.max(p, axis=1, keepdims=True)
    i1 = jnp.min(jnp.where(p == p1, lane, LANES - 1), axis=1, keepdims=True)
    pm = jnp.where(lane == i1, -1.0, p)
    p2 = jnp.max(pm, axis=1, keepdims=True)
    i2 = jnp.min(jnp.where((pm == p2) & valid, lane, LANES - 1),
                 axis=1, keepdims=True)
    keep = p1 >= THR
    keepf = keep.astype(jnp.float32)
    w1 = p1 * keepf
    w2 = p2 * keepf
    oh1 = ((lane == i1) & valid).astype(jnp.float32)
    oh2 = ((lane == i2) & valid).astype(jnp.float32)
    oh2s = (lane == i2 + E).astype(jnp.float32)
    ohk = (oh1 + oh2s) * keepf

    C = 256
    r_i = lax.broadcasted_iota(jnp.int32, (C, C), 0)
    c_i = lax.broadcasted_iota(jnp.int32, (C, C), 1)
    tril = (c_i < r_i).astype(jnp.float32)
    carry = jnp.zeros((1, LANES), jnp.float32)
    cums = []
    for ci in range(T // C):
        blk = lax.slice_in_dim(ohk, ci * C, (ci + 1) * C, axis=0)
        cums.append(jnp.dot(tril, blk, preferred_element_type=jnp.float32)
                    + carry)
        carry = carry + jnp.sum(blk, axis=0, keepdims=True)
    cum = jnp.concatenate(cums, axis=0)
    tot = carry

    a_i = lax.broadcasted_iota(jnp.int32, (LANES, LANES), 0)
    e_i = lax.broadcasted_iota(jnp.int32, (LANES, LANES), 1)
    m_count = (((a_i == e_i) | (a_i == e_i + E)) & (e_i < E)).astype(jnp.float32)
    m_prefix = ((a_i < e_i) & (a_i < E) & (e_i < E)).astype(jnp.float32)
    tot8 = jnp.broadcast_to(tot, (8, LANES))
    count8 = jnp.dot(tot8, m_count, preferred_element_type=jnp.float32)
    padded8 = jnp.floor((count8 + (B - 1)) * (1.0 / B)) * B
    po8 = jnp.dot(padded8, m_prefix, preferred_element_type=jnp.float32)
    po = lax.slice_in_dim(po8, 0, 1, axis=0)
    countr = lax.slice_in_dim(count8, 0, 1, axis=0)

    def lsum(mat, oh):
        return jnp.sum(mat * oh, axis=1, keepdims=True)

    po_b = jnp.broadcast_to(po, (T, LANES))
    tot_b = jnp.broadcast_to(tot, (T, LANES))
    slot1 = lsum(po_b, oh1) + lsum(cum, oh1)
    slot2 = lsum(po_b, oh2) + lsum(tot_b, oh2) + lsum(cum, oh2s)
    s1 = slot1.astype(jnp.int32)
    s2 = slot2.astype(jnp.int32)
    tok = lax.broadcasted_iota(jnp.int32, (T, 1), 0)
    ss1 = jnp.where(keep, s1, CAP + (tok * 2) % DUMP)
    ss2 = jnp.where(keep, s2, CAP + (tok * 2 + 1) % DUMP)

    w_ref[...] = jnp.where(lane < 16, w1, jnp.where(lane < 32, w2, 0.0))
    s_ref[...] = jnp.where(lane == 0, ss1, jnp.where(lane == 1, ss2, 0))

    bl = lax.broadcasted_iota(jnp.int32, (NB, LANES), 1)
    brow = lax.broadcasted_iota(jnp.int32, (NB, LANES), 0)
    bstart = (brow * B).astype(jnp.float32)
    po_nb = jnp.broadcast_to(po, (NB, LANES))
    cnt_nb = jnp.broadcast_to(countr, (NB, LANES))
    le = ((po_nb <= bstart) & (bl < E)).astype(jnp.float32)
    be = (jnp.sum(le, axis=1, keepdims=True) - 1.0).astype(jnp.int32)
    ohbe = (bl == be).astype(jnp.float32)
    bend = jnp.sum((po_nb + cnt_nb) * ohbe, axis=1, keepdims=True)
    ba = (lax.slice_in_dim(bstart, 0, 1, axis=1) < bend).astype(jnp.int32)
    b_ref[...] = jnp.where(bl == 0, be, jnp.where(bl == 1, ba, 0))


_router = pl.pallas_call(
    _router_body,
    grid=(1,),
    in_specs=[
        pl.BlockSpec((T, H), lambda i: (0, 0)),
        pl.BlockSpec((H, E), lambda i: (0, 0)),
    ],
    out_specs=[
        pl.BlockSpec((T, LANES), lambda i: (0, 0)),
        pl.BlockSpec((T, LANES), lambda i: (0, 0)),
        pl.BlockSpec((NB, LANES), lambda i: (0, 0)),
    ],
    out_shape=[
        jax.ShapeDtypeStruct((T, LANES), jnp.float32),
        jax.ShapeDtypeStruct((T, LANES), jnp.int32),
        jax.ShapeDtypeStruct((NB, LANES), jnp.int32),
    ],
)


RING = 5


def _ffn_body(be_ref, ba_ref, xs_hbm, wg_hbm, wu_hbm, wd_hbm, zs_hbm,
              wg_sc, wu_sc, wd_sc, ring, zbuf, zero_buf,
              wsems, lsems, osems):
    def wcp(ei):
        return (pltpu.make_async_copy(wg_hbm.at[ei], wg_sc.at[ei],
                                      wsems.at[ei]),
                pltpu.make_async_copy(wu_hbm.at[ei], wu_sc.at[ei],
                                      wsems.at[ei]),
                pltpu.make_async_copy(wd_hbm.at[ei], wd_sc.at[ei],
                                      wsems.at[ei]))

    def ld(blk, slot):
        return pltpu.make_async_copy(xs_hbm.at[pl.ds(blk * B, B)],
                                     ring.at[slot], lsems.at[slot])

    for blk0 in range(RING - 1):
        ld(blk0, blk0).start()
    for cp in wcp(0):
        cp.start()
    for cp in wcp(1):
        cp.start()
    zero_buf[...] = jnp.zeros((B, H), jnp.float32)

    def loop(blk, elast):
        slot = lax.rem(blk, RING)
        oslot = lax.rem(blk, 2)
        nblk = blk + (RING - 1)

        @pl.when(nblk < NB)
        def _():
            ld(nblk, lax.rem(nblk, RING)).start()

        e = be_ref[blk]
        for j in range(2, E):
            @pl.when((elast < j - 2) & (j - 2 <= e))
            def _(j=j):
                for cp in wcp(j):
                    cp.start()

        ld(blk, slot).wait()
        for ei in range(E):
            @pl.when((elast < ei) & (ei <= e))
            def _(ei=ei):
                for cp in wcp(ei):
                    cp.wait()

        @pl.when(blk >= 2)
        def _():
            pltpu.make_async_copy(zbuf.at[0], zs_hbm.at[pl.ds(0, B)],
                                  osems.at[oslot]).wait()

        act = ba_ref[blk]

        @pl.when(act == 1)
        def _compute():
            xb = ring[slot].astype(jnp.bfloat16)
            wg = wg_sc[e].astype(jnp.bfloat16)
            wu = wu_sc[e].astype(jnp.bfloat16)
            g = jnp.dot(xb, wg, preferred_element_type=jnp.float32)
            u = jnp.dot(xb, wu, preferred_element_type=jnp.float32)
            h = (g * lax.logistic(g)) * u
            zbuf[oslot] = jnp.dot(h.astype(jnp.bfloat16),
                                  wd_sc[e].astype(jnp.bfloat16),
                                  preferred_element_type=jnp.float32)
            pltpu.make_async_copy(zbuf.at[oslot],
                                  zs_hbm.at[pl.ds(blk * B, B)],
                                  osems.at[oslot]).start()

        @pl.when(act == 0)
        def _zero():
            pltpu.make_async_copy(zero_buf, zs_hbm.at[pl.ds(blk * B, B)],
                                  osems.at[oslot]).start()

        return e

    lax.fori_loop(0, NB, loop, jnp.int32(-1))
    pltpu.make_async_copy(zbuf.at[0], zs_hbm.at[pl.ds(0, B)],
                          osems.at[0]).wait()
    pltpu.make_async_copy(zbuf.at[0], zs_hbm.at[pl.ds(0, B)],
                          osems.at[1]).wait()


_grouped_ffn = pl.pallas_call(
    _ffn_body,
    grid_spec=pltpu.PrefetchScalarGridSpec(
        num_scalar_prefetch=2,
        grid=(1,),
        in_specs=[
            pl.BlockSpec(memory_space=pl.ANY),
            pl.BlockSpec(memory_space=pl.ANY),
            pl.BlockSpec(memory_space=pl.ANY),
            pl.BlockSpec(memory_space=pl.ANY),
        ],
        out_specs=pl.BlockSpec(memory_space=pl.ANY),
        scratch_shapes=[
            pltpu.VMEM((E, H, FF), jnp.float32),
            pltpu.VMEM((E, H, FF), jnp.float32),
            pltpu.VMEM((E, FF, H), jnp.float32),
            pltpu.VMEM((RING, B, H), jnp.float32),
            pltpu.VMEM((2, B, H), jnp.float32),
            pltpu.VMEM((B, H), jnp.float32),
            pltpu.SemaphoreType.DMA((E,)),
            pltpu.SemaphoreType.DMA((RING,)),
            pltpu.SemaphoreType.DMA((2,)),
        ],
    ),
    out_shape=jax.ShapeDtypeStruct((XS_ROWS, H), jnp.float32),
)


def _shared_body(x_ref, wg_ref, wu_ref, wd_ref, o_ref):
    xb = x_ref[...].astype(jnp.bfloat16)
    g = jnp.dot(xb, wg_ref[...].astype(jnp.bfloat16),
                preferred_element_type=jnp.float32)
    u = jnp.dot(xb, wu_ref[...].astype(jnp.bfloat16),
                preferred_element_type=jnp.float32)
    h = (g * lax.logistic(g)) * u
    o_ref[...] = jnp.dot(h.astype(jnp.bfloat16),
                         wd_ref[...].astype(jnp.bfloat16),
                         preferred_element_type=jnp.float32)


_shared_ffn = pl.pallas_call(
    _shared_body,
    grid=(2,),
    in_specs=[
        pl.BlockSpec((T // 2, H), lambda i: (i, 0)),
        pl.BlockSpec((H, SFF), lambda i: (0, 0)),
        pl.BlockSpec((H, SFF), lambda i: (0, 0)),
        pl.BlockSpec((SFF, H), lambda i: (0, 0)),
    ],
    out_specs=pl.BlockSpec((T // 2, H), lambda i: (i, 0)),
    out_shape=jax.ShapeDtypeStruct((T, H), jnp.float32),
)


def _extract_columns(sblk_v, col_a, col_b, idx0_v, idx1_v, n):
    lanei = lax.iota(jnp.int32, 16)
    for g in range(n // 16):
        v0 = jnp.zeros((16,), jnp.int32)
        v1 = jnp.zeros((16,), jnp.int32)
        for i in range(16):
            chunk = sblk_v[pl.ds((g * 16 + i) * LANES, 16)]
            v0 = jnp.where(lanei == i, chunk[col_a], v0)
            v1 = jnp.where(lanei == i, chunk[col_b], v1)
        idx0_v[pl.ds(g * 16, 16)] = v0
        idx1_v[pl.ds(g * 16, 16)] = v1


def _sc_dispatch_body(x_hbm, s_hbm, xs_hbm, sblk_v, idx0_v, idx1_v, rows_v,
                      sem0, sem1):
    wid = lax.axis_index("s") * 2 + lax.axis_index("c")
    base = wid * TW
    cpr = pltpu.async_copy(x_hbm.at[pl.ds(base, TW)], rows_v, sem0)
    pltpu.sync_copy(s_hbm.at[pl.ds(base * LANES, TW * LANES)], sblk_v)
    _extract_columns(sblk_v, 0, 1, idx0_v, idx1_v, TW)
    cpr.wait()
    cp0 = pltpu.async_copy(rows_v, xs_hbm.at[idx0_v], sem0)
    cp1 = pltpu.async_copy(rows_v, xs_hbm.at[idx1_v], sem1)
    cp0.wait()
    cp1.wait()


def _sc_combine_body(zs_hbm, sh_hbm, w_hbm, s_hbm, y_hbm,
                     wblk_v, sblk_v, idx0_v, idx1_v, acc_v,
                     zg0a_v, zg1a_v, zg0b_v, zg1b_v,
                     sem_s, sem_a, sem_b, sem_y):
    wid = lax.axis_index("s") * 2 + lax.axis_index("c")
    base = wid * TW
    cps = pltpu.async_copy(sh_hbm.at[pl.ds(base, TW)], acc_v, sem_s)
    pltpu.sync_copy(s_hbm.at[pl.ds(base * LANES, TW * LANES)], sblk_v)
    pltpu.sync_copy(w_hbm.at[pl.ds(base * LANES, TW * LANES)], wblk_v)
    _extract_columns(sblk_v, 0, 1, idx0_v, idx1_v, TW)

    nch = TW // CH
    bufs = [(zg0a_v, zg1a_v, sem_a), (zg0b_v, zg1b_v, sem_b)]

    def start(ch):
        zg0, zg1, sem = bufs[ch % 2]
        d0 = pltpu.async_copy(zs_hbm.at[idx0_v.at[pl.ds(ch * CH, CH)]], zg0,
                              sem)
        d1 = pltpu.async_copy(zs_hbm.at[idx1_v.at[pl.ds(ch * CH, CH)]], zg1,
                              sem)
        return d0, d1

    pending = start(0)
    cps.wait()
    ywaits = []
    for ch in range(nch):
        nxt = start(ch + 1) if ch + 1 < nch else None
        pending[0].wait()
        pending[1].wait()
        zg0, zg1, _ = bufs[ch % 2]

        def row_body(r, carry, ch=ch, zg0=zg0, zg1=zg1):
            rr = ch * CH + r
            a0 = wblk_v[pl.ds(rr * LANES, 16)]
            a1 = wblk_v[pl.ds(rr * LANES + 16, 16)]
            for j in range(H // 16):
                sl = pl.ds(j * 16, 16)
                plsc.addupdate(acc_v.at[rr, sl], zg0[r, sl] * a0
                               + zg1[r, sl] * a1)
            return carry

        lax.fori_loop(0, CH, row_body, 0)
        ywaits.append(pltpu.async_copy(
            acc_v.at[pl.ds(ch * CH, CH)],
            y_hbm.at[pl.ds(base + ch * CH, CH)], sem_y))
        pending = nxt
    for yd in ywaits:
        yd.wait()


@functools.lru_cache(maxsize=1)
def _sc_kernels():
    mesh = plsc.VectorSubcoreMesh(core_axis_name="c", subcore_axis_name="s")
    dispatch = pl.kernel(
        _sc_dispatch_body,
        out_type=jax.ShapeDtypeStruct((XS_ROWS, H), jnp.float32),
        mesh=mesh,
        scratch_types=[
            pltpu.VMEM((TW * LANES,), jnp.int32),
            pltpu.VMEM((TW,), jnp.int32),
            pltpu.VMEM((TW,), jnp.int32),
            pltpu.VMEM((TW, H), jnp.float32),
            pltpu.SemaphoreType.DMA,
            pltpu.SemaphoreType.DMA,
        ],
    )
    combine = pl.kernel(
        _sc_combine_body,
        out_type=jax.ShapeDtypeStruct((T, H), jnp.float32),
        mesh=mesh,
        scratch_types=[
            pltpu.VMEM((TW * LANES,), jnp.float32),
            pltpu.VMEM((TW * LANES,), jnp.int32),
            pltpu.VMEM((TW,), jnp.int32),
            pltpu.VMEM((TW,), jnp.int32),
            pltpu.VMEM((TW, H), jnp.float32),
            pltpu.VMEM((CH, H), jnp.float32),
            pltpu.VMEM((CH, H), jnp.float32),
            pltpu.VMEM((CH, H), jnp.float32),
            pltpu.VMEM((CH, H), jnp.float32),
            pltpu.SemaphoreType.DMA,
            pltpu.SemaphoreType.DMA,
            pltpu.SemaphoreType.DMA,
            pltpu.SemaphoreType.DMA,
        ],
    )
    return dispatch, combine


def kernel(hidden_states, Wr, Wg, Wu, Wd, Wg_s, Wu_s, Wd_s):
    x = hidden_states
    wout, sout, bout = _router(x, Wr)
    be = bout[:, 0]
    ba = bout[:, 1]
    sc_dispatch, sc_combine = _sc_kernels()
    sflat = sout.reshape(-1)
    xs = sc_dispatch(x, sflat)
    zs = _grouped_ffn(be, ba, xs, Wg, Wu, Wd)
    sh = _shared_ffn(x, Wg_s, Wu_s, Wd_s)
    return sc_combine(zs, sh, wout.reshape(-1), sflat)


# speedup vs baseline: 1.2204x; 1.0008x over previous
import functools

import jax
import jax.numpy as jnp
from jax import lax
from jax.experimental import pallas as pl
from jax.experimental.pallas import tpu as pltpu
from jax.experimental.pallas import tpu_sc as plsc

H = 768
E = 8
FF = 512
SFF = 1024
THR = 0.2
T = 2048

B = 256
CAP = T * 2 + E * B
DUMP = 256
XS_ROWS = CAP + DUMP
NB = XS_ROWS // B

NW = 32
TW = T // NW
CH = 16
LANES = 128


def _router_body(x_ref, wr_ref, w_ref, s_ref, b_ref):
    xb = x_ref[...]
    wr = jnp.concatenate(
        [wr_ref[...], jnp.zeros((H, LANES - E), jnp.float32)], axis=1)
    logits = jnp.dot(xb, wr, preferred_element_type=jnp.float32)
    lane = lax.broadcasted_iota(jnp.int32, (T, LANES), 1)
    valid = lane < E
    lg = jnp.where(valid, logits, -1e30)
    m = jnp.max(lg, axis=1, keepdims=True)
    ex = jnp.where(valid, jnp.exp(lg - m), 0.0)
    z = jnp.sum(ex, axis=1, keepdims=True)
    p = ex / z
    p1 = jnp.max(p, axis=1, keepdims=True)
    i1 = jnp.min(jnp.where(p == p1, lane, LANES - 1), axis=1, keepdims=True)
    pm = jnp.where(lane == i1, -1.0, p)
    p2 = jnp.max(pm, axis=1, keepdims=True)
    i2 = jnp.min(jnp.where((pm == p2) & valid, lane, LANES - 1),
                 axis=1, keepdims=True)
    keep = p1 >= THR
    keepf = keep.astype(jnp.float32)
    w1 = p1 * keepf
    w2 = p2 * keepf
    oh1 = ((lane == i1) & valid).astype(jnp.float32)
    oh2 = ((lane == i2) & valid).astype(jnp.float32)
    oh2s = (lane == i2 + E).astype(jnp.float32)
    ohk = (oh1 + oh2s) * keepf

    C = 256
    r_i = lax.broadcasted_iota(jnp.int32, (C, C), 0)
    c_i = lax.broadcasted_iota(jnp.int32, (C, C), 1)
    tril = (c_i < r_i).astype(jnp.float32)
    carry = jnp.zeros((1, LANES), jnp.float32)
    cums = []
    for ci in range(T // C):
        blk = lax.slice_in_dim(ohk, ci * C, (ci + 1) * C, axis=0)
        cums.append(jnp.dot(tril, blk, preferred_element_type=jnp.float32)
                    + carry)
        carry = carry + jnp.sum(blk, axis=0, keepdims=True)
    cum = jnp.concatenate(cums, axis=0)
    tot = carry

    a_i = lax.broadcasted_iota(jnp.int32, (LANES, LANES), 0)
    e_i = lax.broadcasted_iota(jnp.int32, (LANES, LANES), 1)
    m_count = (((a_i == e_i) | (a_i == e_i + E)) & (e_i < E)).astype(jnp.float32)
    m_prefix = ((a_i < e_i) & (a_i < E) & (e_i < E)).astype(jnp.float32)
    tot8 = jnp.broadcast_to(tot, (8, LANES))
    count8 = jnp.dot(tot8, m_count, preferred_element_type=jnp.float32)
    padded8 = jnp.floor((count8 + (B - 1)) * (1.0 / B)) * B
    po8 = jnp.dot(padded8, m_prefix, preferred_element_type=jnp.float32)
    po = lax.slice_in_dim(po8, 0, 1, axis=0)
    countr = lax.slice_in_dim(count8, 0, 1, axis=0)

    def lsum(mat, oh):
        return jnp.sum(mat * oh, axis=1, keepdims=True)

    po_b = jnp.broadcast_to(po, (T, LANES))
    tot_b = jnp.broadcast_to(tot, (T, LANES))
    slot1 = lsum(po_b, oh1) + lsum(cum, oh1)
    slot2 = lsum(po_b, oh2) + lsum(tot_b, oh2) + lsum(cum, oh2s)
    s1 = slot1.astype(jnp.int32)
    s2 = slot2.astype(jnp.int32)
    tok = lax.broadcasted_iota(jnp.int32, (T, 1), 0)
    ss1 = jnp.where(keep, s1, CAP + (tok * 2) % DUMP)
    ss2 = jnp.where(keep, s2, CAP + (tok * 2 + 1) % DUMP)

    w_ref[...] = jnp.where(lane < 16, w1, jnp.where(lane < 32, w2, 0.0))
    s_ref[...] = jnp.where(lane == 0, ss1, jnp.where(lane == 1, ss2, 0))

    bl = lax.broadcasted_iota(jnp.int32, (NB, LANES), 1)
    brow = lax.broadcasted_iota(jnp.int32, (NB, LANES), 0)
    bstart = (brow * B).astype(jnp.float32)
    po_nb = jnp.broadcast_to(po, (NB, LANES))
    cnt_nb = jnp.broadcast_to(countr, (NB, LANES))
    le = ((po_nb <= bstart) & (bl < E)).astype(jnp.float32)
    be = (jnp.sum(le, axis=1, keepdims=True) - 1.0).astype(jnp.int32)
    ohbe = (bl == be).astype(jnp.float32)
    bend = jnp.sum((po_nb + cnt_nb) * ohbe, axis=1, keepdims=True)
    ba = (lax.slice_in_dim(bstart, 0, 1, axis=1) < bend).astype(jnp.int32)
    b_ref[...] = jnp.where(bl == 0, be, jnp.where(bl == 1, ba, 0))


_router = pl.pallas_call(
    _router_body,
    grid=(1,),
    in_specs=[
        pl.BlockSpec((T, H), lambda i: (0, 0)),
        pl.BlockSpec((H, E), lambda i: (0, 0)),
    ],
    out_specs=[
        pl.BlockSpec((T, LANES), lambda i: (0, 0)),
        pl.BlockSpec((T, LANES), lambda i: (0, 0)),
        pl.BlockSpec((NB, LANES), lambda i: (0, 0)),
    ],
    out_shape=[
        jax.ShapeDtypeStruct((T, LANES), jnp.float32),
        jax.ShapeDtypeStruct((T, LANES), jnp.int32),
        jax.ShapeDtypeStruct((NB, LANES), jnp.int32),
    ],
)


RING = 5


def _ffn_body(be_ref, ba_ref, xs_hbm, wg_hbm, wu_hbm, wd_hbm, zs_hbm,
              wg_sc, wu_sc, wd_sc, ring, zbuf, zero_buf,
              wsems, lsems, osems):
    def wcp(ei):
        return (pltpu.make_async_copy(wg_hbm.at[ei], wg_sc.at[ei],
                                      wsems.at[ei]),
                pltpu.make_async_copy(wu_hbm.at[ei], wu_sc.at[ei],
                                      wsems.at[ei]),
                pltpu.make_async_copy(wd_hbm.at[ei], wd_sc.at[ei],
                                      wsems.at[ei]))

    def ld(blk, slot):
        return pltpu.make_async_copy(xs_hbm.at[pl.ds(blk * B, B)],
                                     ring.at[slot], lsems.at[slot])

    for blk0 in range(RING - 1):
        ld(blk0, blk0).start()
    for cp in wcp(0):
        cp.start()
    for cp in wcp(1):
        cp.start()
    zero_buf[...] = jnp.zeros((B, H), jnp.float32)

    def loop(blk, elast):
        slot = lax.rem(blk, RING)
        oslot = lax.rem(blk, 2)
        nblk = blk + (RING - 1)

        @pl.when(nblk < NB)
        def _():
            ld(nblk, lax.rem(nblk, RING)).start()

        e = be_ref[blk]
        for j in range(2, E):
            @pl.when((elast < j - 2) & (j - 2 <= e))
            def _(j=j):
                for cp in wcp(j):
                    cp.start()

        ld(blk, slot).wait()
        for ei in range(E):
            @pl.when((elast < ei) & (ei <= e))
            def _(ei=ei):
                for cp in wcp(ei):
                    cp.wait()

        @pl.when(blk >= 2)
        def _():
            pltpu.make_async_copy(zbuf.at[0], zs_hbm.at[pl.ds(0, B)],
                                  osems.at[oslot]).wait()

        act = ba_ref[blk]

        @pl.when(act == 1)
        def _compute():
            xb = ring[slot].astype(jnp.bfloat16)
            wg = wg_sc[e].astype(jnp.bfloat16)
            wu = wu_sc[e].astype(jnp.bfloat16)
            g = jnp.dot(xb, wg, preferred_element_type=jnp.float32)
            u = jnp.dot(xb, wu, preferred_element_type=jnp.float32)
            h = (g * lax.logistic(g)) * u
            zbuf[oslot] = jnp.dot(h.astype(jnp.bfloat16),
                                  wd_sc[e].astype(jnp.bfloat16),
                                  preferred_element_type=jnp.float32)
            pltpu.make_async_copy(zbuf.at[oslot],
                                  zs_hbm.at[pl.ds(blk * B, B)],
                                  osems.at[oslot]).start()

        @pl.when(act == 0)
        def _zero():
            pltpu.make_async_copy(zero_buf, zs_hbm.at[pl.ds(blk * B, B)],
                                  osems.at[oslot]).start()

        return e

    lax.fori_loop(0, NB, loop, jnp.int32(-1))
    pltpu.make_async_copy(zbuf.at[0], zs_hbm.at[pl.ds(0, B)],
                          osems.at[0]).wait()
    pltpu.make_async_copy(zbuf.at[0], zs_hbm.at[pl.ds(0, B)],
                          osems.at[1]).wait()


_grouped_ffn = pl.pallas_call(
    _ffn_body,
    grid_spec=pltpu.PrefetchScalarGridSpec(
        num_scalar_prefetch=2,
        grid=(1,),
        in_specs=[
            pl.BlockSpec(memory_space=pl.ANY),
            pl.BlockSpec(memory_space=pl.ANY),
            pl.BlockSpec(memory_space=pl.ANY),
            pl.BlockSpec(memory_space=pl.ANY),
        ],
        out_specs=pl.BlockSpec(memory_space=pl.ANY),
        scratch_shapes=[
            pltpu.VMEM((E, H, FF), jnp.float32),
            pltpu.VMEM((E, H, FF), jnp.float32),
            pltpu.VMEM((E, FF, H), jnp.float32),
            pltpu.VMEM((RING, B, H), jnp.float32),
            pltpu.VMEM((2, B, H), jnp.float32),
            pltpu.VMEM((B, H), jnp.float32),
            pltpu.SemaphoreType.DMA((E,)),
            pltpu.SemaphoreType.DMA((RING,)),
            pltpu.SemaphoreType.DMA((2,)),
        ],
    ),
    out_shape=jax.ShapeDtypeStruct((XS_ROWS, H), jnp.float32),
)


def _shared_body(x_ref, wg_ref, wu_ref, wd_ref, o_ref):
    xb = x_ref[...].astype(jnp.bfloat16)
    g = jnp.dot(xb, wg_ref[...].astype(jnp.bfloat16),
                preferred_element_type=jnp.float32)
    u = jnp.dot(xb, wu_ref[...].astype(jnp.bfloat16),
                preferred_element_type=jnp.float32)
    h = (g * lax.logistic(g)) * u
    o_ref[...] = jnp.dot(h.astype(jnp.bfloat16),
                         wd_ref[...].astype(jnp.bfloat16),
                         preferred_element_type=jnp.float32)


_shared_ffn = pl.pallas_call(
    _shared_body,
    grid=(2,),
    in_specs=[
        pl.BlockSpec((T // 2, H), lambda i: (i, 0)),
        pl.BlockSpec((H, SFF), lambda i: (0, 0)),
        pl.BlockSpec((H, SFF), lambda i: (0, 0)),
        pl.BlockSpec((SFF, H), lambda i: (0, 0)),
    ],
    out_specs=pl.BlockSpec((T // 2, H), lambda i: (i, 0)),
    out_shape=jax.ShapeDtypeStruct((T, H), jnp.float32),
)


def _extract_columns(sblk_v, col_a, col_b, idx0_v, idx1_v, n):
    lanei = lax.iota(jnp.int32, 16)
    for g in range(n // 16):
        v0 = jnp.zeros((16,), jnp.int32)
        v1 = jnp.zeros((16,), jnp.int32)
        for i in range(16):
            chunk = sblk_v[g * 16 + i, 0:16]
            v0 = jnp.where(lanei == i, chunk[col_a], v0)
            v1 = jnp.where(lanei == i, chunk[col_b], v1)
        idx0_v[pl.ds(g * 16, 16)] = v0
        idx1_v[pl.ds(g * 16, 16)] = v1


def _sc_dispatch_body(x_hbm, s_hbm, xs_hbm, sblk_v, idx0_v, idx1_v, rows_v,
                      sem0, sem1):
    wid = lax.axis_index("s") * 2 + lax.axis_index("c")
    base = wid * TW
    cpr = pltpu.async_copy(x_hbm.at[pl.ds(base, TW)], rows_v, sem0)
    pltpu.sync_copy(s_hbm.at[pl.ds(base, TW)], sblk_v)
    _extract_columns(sblk_v, 0, 1, idx0_v, idx1_v, TW)
    cpr.wait()
    cp0 = pltpu.async_copy(rows_v, xs_hbm.at[idx0_v], sem0)
    cp1 = pltpu.async_copy(rows_v, xs_hbm.at[idx1_v], sem1)
    cp0.wait()
    cp1.wait()


def _sc_combine_body(zs_hbm, sh_hbm, w_hbm, s_hbm, y_hbm,
                     wblk_v, sblk_v, idx0_v, idx1_v, acc_v,
                     zg0a_v, zg1a_v, zg0b_v, zg1b_v,
                     sem_s, sem_a, sem_b, sem_y):
    wid = lax.axis_index("s") * 2 + lax.axis_index("c")
    base = wid * TW
    cps = pltpu.async_copy(sh_hbm.at[pl.ds(base, TW)], acc_v, sem_s)
    pltpu.sync_copy(s_hbm.at[pl.ds(base, TW)], sblk_v)
    pltpu.sync_copy(w_hbm.at[pl.ds(base, TW)], wblk_v)
    _extract_columns(sblk_v, 0, 1, idx0_v, idx1_v, TW)

    nch = TW // CH
    bufs = [(zg0a_v, zg1a_v, sem_a), (zg0b_v, zg1b_v, sem_b)]

    def start(ch):
        zg0, zg1, sem = bufs[ch % 2]
        d0 = pltpu.async_copy(zs_hbm.at[idx0_v.at[pl.ds(ch * CH, CH)]], zg0,
                              sem)
        d1 = pltpu.async_copy(zs_hbm.at[idx1_v.at[pl.ds(ch * CH, CH)]], zg1,
                              sem)
        return d0, d1

    pending = start(0)
    cps.wait()
    ywaits = []
    for ch in range(nch):
        nxt = start(ch + 1) if ch + 1 < nch else None
        pending[0].wait()
        pending[1].wait()
        zg0, zg1, _ = bufs[ch % 2]

        def row_body(r, carry, ch=ch, zg0=zg0, zg1=zg1):
            rr = ch * CH + r
            a0 = wblk_v[rr, 0:16]
            a1 = wblk_v[rr, 16:32]
            for j in range(H // 16):
                sl = pl.ds(j * 16, 16)
                plsc.addupdate(acc_v.at[rr, sl], zg0[r, sl] * a0
                               + zg1[r, sl] * a1)
            return carry

        lax.fori_loop(0, CH, row_body, 0)
        ywaits.append(pltpu.async_copy(
            acc_v.at[pl.ds(ch * CH, CH)],
            y_hbm.at[pl.ds(base + ch * CH, CH)], sem_y))
        pending = nxt
    for yd in ywaits:
        yd.wait()


@functools.lru_cache(maxsize=1)
def _sc_kernels():
    mesh = plsc.VectorSubcoreMesh(core_axis_name="c", subcore_axis_name="s")
    dispatch = pl.kernel(
        _sc_dispatch_body,
        out_type=jax.ShapeDtypeStruct((XS_ROWS, H), jnp.float32),
        mesh=mesh,
        scratch_types=[
            pltpu.VMEM((TW, LANES), jnp.int32),
            pltpu.VMEM((TW,), jnp.int32),
            pltpu.VMEM((TW,), jnp.int32),
            pltpu.VMEM((TW, H), jnp.float32),
            pltpu.SemaphoreType.DMA,
            pltpu.SemaphoreType.DMA,
        ],
    )
    combine = pl.kernel(
        _sc_combine_body,
        out_type=jax.ShapeDtypeStruct((T, H), jnp.float32),
        mesh=mesh,
        scratch_types=[
            pltpu.VMEM((TW, LANES), jnp.float32),
            pltpu.VMEM((TW, LANES), jnp.int32),
            pltpu.VMEM((TW,), jnp.int32),
            pltpu.VMEM((TW,), jnp.int32),
            pltpu.VMEM((TW, H), jnp.float32),
            pltpu.VMEM((CH, H), jnp.float32),
            pltpu.VMEM((CH, H), jnp.float32),
            pltpu.VMEM((CH, H), jnp.float32),
            pltpu.VMEM((CH, H), jnp.float32),
            pltpu.SemaphoreType.DMA,
            pltpu.SemaphoreType.DMA,
            pltpu.SemaphoreType.DMA,
            pltpu.SemaphoreType.DMA,
        ],
    )
    return dispatch, combine


def kernel(hidden_states, Wr, Wg, Wu, Wd, Wg_s, Wu_s, Wd_s):
    x = hidden_states
    wout, sout, bout = _router(x, Wr)
    be = bout[:, 0]
    ba = bout[:, 1]
    sc_dispatch, sc_combine = _sc_kernels()
    xs = sc_dispatch(x, sout)
    zs = _grouped_ffn(be, ba, xs, Wg, Wu, Wd)
    sh = _shared_ffn(x, Wg_s, Wu_s, Wd_s)
    return sc_combine(zs, sh, wout, sout)
